# baseline (speedup 1.0000x reference)
import functools

import jax
import jax.numpy as jnp
from jax import lax
from jax.experimental import pallas as pl
from jax.experimental.pallas import tpu as pltpu
from jax.experimental.pallas import tpu_sc as plsc

D_MODEL = 64
SCALE = 0.125
NBUF = 4
C = 128
U = 4


@jax.jit
def _embed(x, table):
    idx = x.reshape(-1)
    B = idx.shape[0]
    V = table.shape[0]
    t2 = table.reshape(V // 2, 2 * D_MODEL)

    info = plsc.get_sparse_core_info()
    NC, NS = info.num_cores, info.num_subcores
    NW = NC * NS
    b_per_w = B // NW
    assert b_per_w * NW == B
    n_chunks = b_per_w // C
    assert n_chunks * C == b_per_w
    assert n_chunks >= 6 and (n_chunks - 4) % NBUF == 0
    n_outer = (n_chunks - 4) // NBUF

    mesh = plsc.VectorSubcoreMesh(core_axis_name="c", subcore_axis_name="s")

    @functools.partial(
        pl.kernel,
        mesh=mesh,
        compiler_params=pltpu.CompilerParams(use_tc_tiling_on_sc=False),
        out_type=jax.ShapeDtypeStruct((B, 128), jnp.float32),
        scratch_types=(
            [pltpu.VMEM((b_per_w,), jnp.int32)]
            + [pltpu.VMEM((C,), jnp.int32) for _ in range(NBUF)]
            + [pltpu.VMEM((C, 2 * D_MODEL), jnp.float32) for _ in range(NBUF)]
            + [pltpu.SemaphoreType.DMA for _ in range(2 * NBUF)]
        ),
    )
    def sc_kernel(t2_hbm, idx_hbm, out_hbm, idx_all, *bufs):
        pair = bufs[:NBUF]
        rows = bufs[NBUF:2 * NBUF]
        gsem = bufs[2 * NBUF:3 * NBUF]
        wsem = bufs[3 * NBUF:]
        wid = lax.axis_index("s") * NC + lax.axis_index("c")
        base = wid * b_per_w

        def prep_and_start_gather(g, b):
            def pbody(k, c):
                sl = pl.ds(k * 16, 16)
                pair[b][sl] = lax.shift_right_logical(
                    idx_all[pl.ds(g * C + k * 16, 16)], 1)
                return c

            lax.fori_loop(0, C // 16, pbody, 0)
            pltpu.make_async_copy(t2_hbm.at[pair[b]], rows[b], gsem[b]).start()

        def wait_gather(b):
            pltpu.make_async_copy(t2_hbm.at[pair[b]], rows[b], gsem[b]).wait()

        def wb_desc(g, b):
            return pltpu.make_async_copy(
                rows[b].at[:, pl.ds(0, D_MODEL)],
                out_hbm.at[pl.ds(base + g * C, C), pl.ds(0, D_MODEL)],
                wsem[b])

        def select_scale(g, b):
            rb = rows[b]

            def sbody(k, c):
                idxv = idx_all[pl.ds(g * C + k * 16, 16)]
                hs = (idxv & 1) * D_MODEL
                for u in range(16):
                    h = hs[u]
                    r = k * 16 + u
                    for j in range(D_MODEL // 16):
                        rb[r, pl.ds(j * 16, 16)] = (
                            rb[r, pl.ds(h + j * 16, 16)] * SCALE)
                return c

            lax.fori_loop(0, C // 16, sbody, 0)

        pltpu.sync_copy(idx_hbm.at[pl.ds(base, b_per_w)], idx_all)

        prep_and_start_gather(0, 0)
        prep_and_start_gather(1, 1)

        for g in (0, 1):
            b = g % NBUF
            wait_gather(b)
            select_scale(g, b)
            wb_desc(g, b).start()
            prep_and_start_gather(g + 2, (g + 2) % NBUF)

        def outer(go, c):
            for k in range(NBUF):
                g = 2 + go * NBUF + k
                b = (2 + k) % NBUF
                b2 = k
                wait_gather(b)
                select_scale(g, b)
                wb_desc(g, b).start()
                wb_desc(g - 2, b2).wait()
                prep_and_start_gather(g + 2, b2)
            return c

        lax.fori_loop(0, n_outer, outer, 0)

        for g in (n_chunks - 2, n_chunks - 1):
            b = g % NBUF
            wait_gather(b)
            select_scale(g, b)
            wb_desc(g, b).start()

        for g in range(n_chunks - NBUF, n_chunks):
            wb_desc(g, g % NBUF).wait()

    out = sc_kernel(t2, idx)
    return out[:, :D_MODEL].reshape(x.shape + (D_MODEL,))


def kernel(x, table):
    return _embed(x, table)
